# Initial kernel scaffold; baseline (speedup 1.0000x reference)
import jax
import jax.numpy as jnp
from jax.experimental import pallas as pl


def kernel(node_labels, adj_e0, adj_e1, adj_e2, adj_e3, label_emb, W_init, edge_W, gru_W, gru_U, gru_b):
    raise NotImplementedError("write your pallas kernel here")


# speedup vs baseline: 1.9836x; 1.9836x over previous
import functools

import jax
import jax.numpy as jnp
from jax import lax
from jax.experimental import pallas as pl
from jax.experimental.pallas import tpu as pltpu
from jax.experimental.pallas import tpu_sc as plsc

H = 64
NT = 4
NDIR = 2 * NT
TIMESTEPS = [7, 1]
NSUB = 32
LANES = 16
B_E = 128
CN = 32


def _ceil_to(x, m):
    return (x + m - 1) // m * m


def _mm_scaled_body(x_ref, w_ref, o_ref, *, scale):
    o_ref[...] = jnp.dot(
        x_ref[...], w_ref[...] * scale, preferred_element_type=jnp.float32
    )


def _tc_matmul(x, w, scale=1.0, block_rows=1000):
    n, k = x.shape
    _, m = w.shape
    br = block_rows if n % block_rows == 0 else n
    return pl.pallas_call(
        functools.partial(_mm_scaled_body, scale=scale),
        grid=(n // br,),
        in_specs=[
            pl.BlockSpec((br, k), lambda i: (i, 0)),
            pl.BlockSpec((k, m), lambda i: (0, 0)),
        ],
        out_specs=pl.BlockSpec((br, m), lambda i: (i, 0)),
        out_shape=jax.ShapeDtypeStruct((n, m), jnp.float32),
    )(x, w)


def _gru_body(agg_ref, h_ref, w_ref, u_ref, b_ref, h0_ref, o_ref, *, add_h0):
    agg = agg_ref[...]
    m = jnp.where(jnp.isfinite(agg), agg, 0.0)
    h = h_ref[...]
    W = w_ref[...]
    U = u_ref[...]
    b = b_ref[...]
    mw = jnp.dot(m, W, preferred_element_type=jnp.float32)
    hu = jnp.dot(h, U[:, : 2 * H], preferred_element_type=jnp.float32)
    zr = mw[:, : 2 * H] + hu + b[:, : 2 * H]
    z = jax.nn.sigmoid(zr[:, :H])
    r = jax.nn.sigmoid(zr[:, H:])
    rhu = jnp.dot(r * h, U[:, 2 * H :], preferred_element_type=jnp.float32)
    n = jnp.tanh(mw[:, 2 * H :] + rhu + b[:, 2 * H :])
    out = (1.0 - z) * h + z * n
    if add_h0:
        out = out + h0_ref[...]
    o_ref[...] = out


def _tc_gru(agg, h, W, U, b, h0, add_h0, block_rows=1000):
    n = h.shape[0]
    br = block_rows if n % block_rows == 0 else n
    return pl.pallas_call(
        functools.partial(_gru_body, add_h0=add_h0),
        grid=(n // br,),
        in_specs=[
            pl.BlockSpec((br, H), lambda i: (i, 0)),
            pl.BlockSpec((br, H), lambda i: (i, 0)),
            pl.BlockSpec((H, 3 * H), lambda i: (0, 0)),
            pl.BlockSpec((H, 3 * H), lambda i: (0, 0)),
            pl.BlockSpec((1, 3 * H), lambda i: (0, 0)),
            pl.BlockSpec((br, H), lambda i: (i, 0)),
        ],
        out_specs=pl.BlockSpec((br, H), lambda i: (i, 0)),
        out_shape=jax.ShapeDtypeStruct((n, H), jnp.float32),
    )(agg, h, W, U, b.reshape(1, 3 * H), h0)


def _sc_worker_id():
    return lax.axis_index("s") * 2 + lax.axis_index("c")


_SC_PARAMS = pltpu.CompilerParams(
    use_tc_tiling_on_sc=False, needs_layout_passes=False
)


def _make_embed_kernel(knode, npad):
    mesh = plsc.VectorSubcoreMesh(core_axis_name="c", subcore_axis_name="s")
    nchunks = knode // CN
    assert knode % CN == 0

    @functools.partial(
        pl.kernel,
        mesh=mesh,
        out_type=jax.ShapeDtypeStruct((npad * H,), jnp.float32),
        compiler_params=_SC_PARAMS,
        scratch_types=[
            pltpu.VMEM((CN * 5,), jnp.int32),
            pltpu.VMEM((CN * 5, H), jnp.float32),
            pltpu.VMEM((CN * H,), jnp.float32),
            pltpu.SemaphoreType.DMA,
        ],
    )
    def embed_kernel(le2_hbm, labels_hbm, out_hbm, lidx, rows, acc, sem):
        w = _sc_worker_id()
        lo = w * knode

        def chunk(c, _):
            base5 = pl.multiple_of((lo + c * CN) * 5, 8)
            pltpu.sync_copy(labels_hbm.at[pl.ds(base5, CN * 5)], lidx)
            pltpu.async_copy(le2_hbm.at[lidx], rows, sem).wait()
            for n in range(CN):
                for k in range(H // LANES):
                    s = rows[n * 5 + 0, pl.ds(k * LANES, LANES)]
                    for j in range(1, 5):
                        s = s + rows[n * 5 + j, pl.ds(k * LANES, LANES)]
                    acc[pl.ds(n * H + k * LANES, LANES)] = s
            base64 = pl.multiple_of((lo + c * CN) * H, 8)
            pltpu.sync_copy(acc, out_hbm.at[pl.ds(base64, CN * H)])
            return 0

        lax.fori_loop(0, nchunks, chunk, 0)

    return embed_kernel


def _make_agg_kernel(knode, npad, n_rows_p2, ep_pad):
    mesh = plsc.VectorSubcoreMesh(core_axis_name="c", subcore_axis_name="s")

    @functools.partial(
        pl.kernel,
        mesh=mesh,
        out_type=jax.ShapeDtypeStruct((npad * H,), jnp.float32),
        compiler_params=_SC_PARAMS,
        scratch_types=[
            pltpu.VMEM((knode * H,), jnp.float32),
            pltpu.VMEM((B_E, H), jnp.float32),
            pltpu.VMEM((B_E,), jnp.int32),
            pltpu.VMEM((B_E,), jnp.int32),
            pltpu.VMEM((48,), jnp.int32),
            pltpu.SemaphoreType.DMA,
        ],
    )
    def agg_kernel(p2_hbm, gs_hbm, doff_hbm, starts_hbm, out_hbm,
                   aggv, msgv, gidx, doffv, sv, sem):
        w = _sc_worker_id()
        lo64 = w * (knode * H)
        iot = lax.iota(jnp.int32, LANES)

        pltpu.sync_copy(starts_hbm, sv)
        s0 = sv[pl.ds(0, 16)]
        s1 = sv[pl.ds(16, 16)]
        s2 = sv[pl.ds(32, 16)]

        def pick(i):
            v = (
                jnp.where(iot == i, s0, 0)
                + jnp.where(iot == i - 16, s1, 0)
                + jnp.where(iot == i - 32, s2, 0)
            )
            return jnp.sum(v)

        start_w = pick(w)
        end_w = pick(w + 1)

        neg = jnp.full((LANES,), -jnp.inf, jnp.float32)

        def initbody(i, _):
            base = i * H
            for k in range(H // LANES):
                aggv[pl.ds(base + k * LANES, LANES)] = neg
            return 0

        lax.fori_loop(0, knode, initbody, 0)

        c0 = start_w // B_E
        c1 = (end_w + (B_E - 1)) // B_E

        def chunk(c, _):
            base_e = pl.multiple_of(c * B_E, B_E)
            pltpu.sync_copy(gs_hbm.at[pl.ds(base_e, B_E)], gidx)
            pltpu.sync_copy(doff_hbm.at[pl.ds(base_e, B_E)], doffv)
            pltpu.async_copy(p2_hbm.at[gidx], msgv, sem).wait()

            def grp(g, _):
                for e in range(LANES):
                    row = g * LANES + e
                    pos = base_e + row
                    valid = (pos >= start_w) & (pos < end_w)
                    mask = jnp.broadcast_to(valid, (LANES,))
                    sel = jnp.full((LANES,), row, jnp.int32)
                    dof = plsc.load_gather(doffv, [sel])
                    loc = jnp.clip(dof - lo64, 0, (knode - 1) * H)
                    for k in range(H // LANES):
                        fidx = loc + (k * LANES) + iot
                        a = plsc.load_gather(aggv, [fidx], mask=mask)
                        mrow = plsc.load_gather(
                            msgv, [sel, (k * LANES) + iot], mask=mask
                        )
                        plsc.store_scatter(
                            aggv, [fidx], jnp.maximum(a, mrow), mask=mask
                        )
                return 0

            lax.fori_loop(0, B_E // LANES, grp, 0)
            return 0

        lax.fori_loop(c0, c1, chunk, 0)
        out_off = pl.multiple_of(w * (knode * H), 8)
        pltpu.sync_copy(aggv, out_hbm.at[pl.ds(out_off, knode * H)])

    return agg_kernel


def kernel(node_labels, adj_e0, adj_e1, adj_e2, adj_e3, label_emb, W_init,
           edge_W, gru_W, gru_U, gru_b):
    n = node_labels.shape[0]
    v = label_emb.shape[0]
    e = adj_e0.shape[0]

    knode = _ceil_to(-(-n // NSUB), 8)
    npad = NSUB * knode

    adjs = [adj_e0, adj_e1, adj_e2, adj_e3]
    g_parts = []
    d_parts = []
    for i in range(NT):
        src = adjs[i][:, 0]
        dst = adjs[i][:, 1]
        g_parts.append(src * NDIR + 2 * i)
        d_parts.append(dst)
        g_parts.append(dst * NDIR + (2 * i + 1))
        d_parts.append(src)
    g = jnp.concatenate(g_parts)
    d = jnp.concatenate(d_parts)
    order = jnp.argsort(d)
    ds = d[order]
    gs = g[order]
    ep = NDIR * e
    ep_pad = _ceil_to(ep, B_E)
    if ep_pad != ep:
        gs = jnp.pad(gs, (0, ep_pad - ep))
        dsp = jnp.pad(ds, (0, ep_pad - ep))
    else:
        dsp = ds
    doff = dsp * H
    bounds = jnp.arange(NSUB + 1, dtype=jnp.int32) * knode
    starts = jnp.searchsorted(ds, bounds, side="left").astype(jnp.int32)
    starts = jnp.pad(starts, (0, 48 - (NSUB + 1)), constant_values=ep)

    labels_flat = node_labels.reshape(-1)
    labels_pad = jnp.pad(labels_flat, (0, npad * 5 - n * 5))

    wcat = [
        jnp.transpose(edge_W[l], (1, 0, 2)).reshape(H, NDIR * H)
        for l in range(2)
    ]

    le2 = _tc_matmul(label_emb, W_init, scale=0.2, block_rows=1000)
    embed = _make_embed_kernel(knode, npad)
    h0 = embed(le2, labels_pad).reshape(npad, H)[:n]

    agg_call = _make_agg_kernel(knode, npad, n * NDIR, ep_pad)

    h = h0
    steps = [(l, t) for l, T in enumerate(TIMESTEPS) for t in range(T)]
    last_of_layer0 = TIMESTEPS[0] - 1
    for si, (l, t) in enumerate(steps):
        p2 = _tc_matmul(h, wcat[l], block_rows=1000).reshape(n * NDIR, H)
        aggf = agg_call(p2, gs, doff, starts).reshape(npad, H)[:n]
        add_h0 = l == 0 and t == last_of_layer0
        h = _tc_gru(aggf, h, gru_W[l], gru_U[l], gru_b[l], h0, add_h0)
    return h
